# baseline (speedup 1.0000x reference)
import functools

import jax
import jax.numpy as jnp
from jax.experimental import pallas as pl
from jax.experimental.pallas import tpu as pltpu

_B, _N, _D = 8, 1024, 128
_H, _HD = 8, 16
_OUT = _H * _HD
_BI = 512
_NI = _N // _BI
_SLOPE = 0.2


def _gat_block_kernel(feat_ref, featb_ref, adj_ref, w_ref, asrc_ref,
                      adst_ref, bias_ref, out_ref):
    i_blk = pl.program_id(1)
    x = feat_ref[0].astype(jnp.bfloat16)
    wb = w_ref[...].astype(jnp.bfloat16)
    xp_f = jnp.dot(x, wb, preferred_element_type=jnp.float32)
    xp = xp_f.astype(jnp.bfloat16)
    asrc_b = asrc_ref[...].astype(jnp.bfloat16)
    adst_b = adst_ref[...].astype(jnp.bfloat16)
    a_src = jnp.dot(xp, asrc_b, preferred_element_type=jnp.float32)
    msrc = jnp.max(a_src, axis=0, keepdims=True)

    adj = adj_ref[0]
    adjb = adj.astype(jnp.bfloat16)

    x_blk = featb_ref[0].astype(jnp.bfloat16)
    xp_blk = jnp.dot(x_blk, wb, preferred_element_type=jnp.float32)
    xp_blk_b = xp_blk.astype(jnp.bfloat16)
    asrc_blk = jnp.dot(xp_blk_b, asrc_b,
                       preferred_element_type=jnp.float32)
    adst_blk = jnp.dot(xp_blk_b, adst_b,
                       preferred_element_type=jnp.float32)

    e_src1 = jnp.exp(a_src)
    e_src2 = jnp.exp(_SLOPE * a_src)
    t_u = adst_blk + msrc
    c_u = jnp.maximum(t_u, _SLOPE * t_u)
    f12_u = jnp.exp(
        jnp.concatenate([adst_blk, _SLOPE * adst_blk], axis=1)
        - jnp.concatenate([c_u, c_u], axis=1))
    f12_t = f12_u.T

    ones_n = jnp.ones((_N, 1), dtype=jnp.bfloat16)
    ones_b = jnp.ones((_BI, 1), dtype=jnp.bfloat16)

    adj_sq = adj_ref[0, pl.ds(i_blk * _BI, _BI), :].astype(jnp.bfloat16)
    kk = jax.lax.broadcasted_iota(jnp.int32, (_BI, _BI), 0)
    ll = jax.lax.broadcasted_iota(jnp.int32, (_BI, _BI), 1)
    eye_b = (kk == ll).astype(jnp.bfloat16)
    adjd = jnp.dot(adj_sq * eye_b, ones_b,
                   preferred_element_type=jnp.float32)

    e12_blk = jnp.exp(
        jnp.concatenate([asrc_blk, _SLOPE * asrc_blk], axis=1))
    q_diag = jnp.maximum(e12_blk[:, :_H] * f12_u[:, :_H],
                         e12_blk[:, _H:] * f12_u[:, _H:])
    w_diag = q_diag * (1.0 - adjd)
    rhs_all = jnp.concatenate([xp, ones_n], axis=1)

    e1b = e_src1.astype(jnp.bfloat16)
    e2b = e_src2.astype(jnp.bfloat16)
    f12b = f12_t.astype(jnp.bfloat16)
    zrow = jnp.zeros((1, _BI), dtype=jnp.bfloat16)

    outs = []
    for h in range(_H):
        lhs2 = jnp.concatenate([e1b[:, h:h + 1], e2b[:, h:h + 1]], axis=1)
        rhs2 = jnp.concatenate([
            jnp.concatenate([f12b[h:h + 1, :], zrow], axis=1),
            jnp.concatenate([zrow, f12b[_H + h:_H + h + 1, :]], axis=1),
        ], axis=0)
        q12 = jnp.dot(lhs2, rhs2,
                      preferred_element_type=jnp.float32).astype(jnp.bfloat16)
        p = jnp.maximum(q12[:, :_BI], q12[:, _BI:]) * adjb
        nd = jax.lax.dot_general(p, rhs_all, (((0,), (0,)), ((), ())),
                                 preferred_element_type=jnp.float32)
        wcol = w_diag[:, h:h + 1]
        num = nd[:, h * _HD:(h + 1) * _HD] \
            + wcol * xp_blk[:, h * _HD:(h + 1) * _HD]
        outs.append(num / (nd[:, _OUT:_OUT + 1] + wcol))

    o = jnp.concatenate(outs, axis=1) + bias_ref[...]
    out_ref[0] = jnp.where(o > 0.0, o, jnp.exp(o) - 1.0)


@jax.jit
def kernel(features_batch, adj_mats_batch, W, att_src, att_dst, bias):
    eye = jnp.eye(_H, dtype=jnp.float32)
    a_src_mat = (att_src[:, :, None] * eye[:, None, :]).reshape(_OUT, _H)
    a_dst_mat = (att_dst[:, :, None] * eye[:, None, :]).reshape(_OUT, _H)
    bias2d = bias.reshape(1, _OUT)

    return pl.pallas_call(
        _gat_block_kernel,
        grid=(_B, _NI),
        in_specs=[
            pl.BlockSpec((1, _N, _D), lambda b, i: (b, 0, 0)),
            pl.BlockSpec((1, _BI, _D), lambda b, i: (b, i, 0)),
            pl.BlockSpec((1, _N, _BI), lambda b, i: (b, 0, i)),
            pl.BlockSpec((_D, _OUT), lambda b, i: (0, 0)),
            pl.BlockSpec((_OUT, _H), lambda b, i: (0, 0)),
            pl.BlockSpec((_OUT, _H), lambda b, i: (0, 0)),
            pl.BlockSpec((1, _OUT), lambda b, i: (0, 0)),
        ],
        out_specs=pl.BlockSpec((1, _BI, _OUT), lambda b, i: (b, i, 0)),
        out_shape=jax.ShapeDtypeStruct((_B, _N, _OUT), jnp.float32),
        compiler_params=pltpu.CompilerParams(
            dimension_semantics=("parallel", "arbitrary")),
    )(features_batch, features_batch, adj_mats_batch, W, a_src_mat,
      a_dst_mat, bias2d)


# speedup vs baseline: 1.0031x; 1.0031x over previous
import jax
import jax.numpy as jnp
from jax.experimental import pallas as pl
from jax.experimental.pallas import tpu as pltpu

_B, _N, _D = 8, 1024, 128
_H, _HD = 8, 16
_OUT = _H * _HD
_BI = 512
_NI = _N // _BI
_SLOPE = 0.2


def _gat_block_kernel(feat_ref, featb_ref, adj_ref, w_ref, asrc_ref,
                      adst_ref, bias_ref, out_ref):
    i_blk = pl.program_id(1)
    x = feat_ref[0].astype(jnp.bfloat16)
    wb = w_ref[...].astype(jnp.bfloat16)
    xp_f = jnp.dot(x, wb, preferred_element_type=jnp.float32)
    xp = xp_f.astype(jnp.bfloat16)
    a_src = jnp.dot(xp_f, asrc_ref[...], preferred_element_type=jnp.float32)
    msrc = jnp.max(a_src, axis=0, keepdims=True)

    adj = adj_ref[0]
    adjb = adj.astype(jnp.bfloat16)

    x_blk = featb_ref[0].astype(jnp.bfloat16)
    xp_blk = jnp.dot(x_blk, wb, preferred_element_type=jnp.float32)
    asrc_blk = jnp.dot(xp_blk, asrc_ref[...],
                       preferred_element_type=jnp.float32)
    adst_blk = jnp.dot(xp_blk, adst_ref[...],
                       preferred_element_type=jnp.float32)

    e_src1 = jnp.exp(a_src)
    e_src2 = jnp.exp(_SLOPE * a_src)
    t_u = adst_blk + msrc
    c_u = jnp.maximum(t_u, _SLOPE * t_u)
    f12_u = jnp.exp(
        jnp.concatenate([adst_blk, _SLOPE * adst_blk], axis=1)
        - jnp.concatenate([c_u, c_u], axis=1))
    f12_t = f12_u.T

    ones_n = jnp.ones((_N, 1), dtype=jnp.bfloat16)
    ones_b = jnp.ones((_BI, 1), dtype=jnp.float32)

    adj_sq = adj_ref[0, pl.ds(i_blk * _BI, _BI), :]
    kk = jax.lax.broadcasted_iota(jnp.int32, (_BI, _BI), 0)
    ll = jax.lax.broadcasted_iota(jnp.int32, (_BI, _BI), 1)
    eye_b = (kk == ll).astype(jnp.float32)
    adjd = jnp.dot(adj_sq * eye_b, ones_b,
                   preferred_element_type=jnp.float32)

    e12_blk = jnp.exp(
        jnp.concatenate([asrc_blk, _SLOPE * asrc_blk], axis=1))
    q_diag = jnp.maximum(e12_blk[:, :_H] * f12_u[:, :_H],
                         e12_blk[:, _H:] * f12_u[:, _H:])
    w_diag = q_diag * (1.0 - adjd)
    rhs_all = jnp.concatenate([xp, ones_n], axis=1)

    e1b = e_src1.astype(jnp.bfloat16)
    e2b = e_src2.astype(jnp.bfloat16)
    f12b = f12_t.astype(jnp.bfloat16)
    zrow = jnp.zeros((1, _BI), dtype=jnp.bfloat16)

    outs = []
    for h in range(_H):
        lhs2 = jnp.concatenate([e1b[:, h:h + 1], e2b[:, h:h + 1]], axis=1)
        rhs2 = jnp.concatenate([
            jnp.concatenate([f12b[h:h + 1, :], zrow], axis=1),
            jnp.concatenate([zrow, f12b[_H + h:_H + h + 1, :]], axis=1),
        ], axis=0)
        q12 = jnp.dot(lhs2, rhs2,
                      preferred_element_type=jnp.float32).astype(jnp.bfloat16)
        p = jnp.maximum(q12[:, :_BI], q12[:, _BI:]) * adjb
        nd = jax.lax.dot_general(p, rhs_all, (((0,), (0,)), ((), ())),
                                 preferred_element_type=jnp.float32)
        wcol = w_diag[:, h:h + 1]
        num = nd[:, h * _HD:(h + 1) * _HD] \
            + wcol * xp_blk[:, h * _HD:(h + 1) * _HD]
        outs.append(num / (nd[:, _OUT:_OUT + 1] + wcol))

    o = jnp.concatenate(outs, axis=1) + bias_ref[...]
    out_ref[0] = jnp.where(o > 0.0, o, jnp.exp(o) - 1.0)


@jax.jit
def kernel(features_batch, adj_mats_batch, W, att_src, att_dst, bias):
    eye = jnp.eye(_H, dtype=jnp.float32)
    a_src_mat = (att_src[:, :, None] * eye[:, None, :]).reshape(_OUT, _H)
    a_dst_mat = (att_dst[:, :, None] * eye[:, None, :]).reshape(_OUT, _H)
    bias2d = bias.reshape(1, _OUT)

    return pl.pallas_call(
        _gat_block_kernel,
        grid=(_B, _NI),
        in_specs=[
            pl.BlockSpec((1, _N, _D), lambda b, i: (b, 0, 0)),
            pl.BlockSpec((1, _BI, _D), lambda b, i: (b, i, 0)),
            pl.BlockSpec((1, _N, _BI), lambda b, i: (b, 0, i)),
            pl.BlockSpec((_D, _OUT), lambda b, i: (0, 0)),
            pl.BlockSpec((_OUT, _H), lambda b, i: (0, 0)),
            pl.BlockSpec((_OUT, _H), lambda b, i: (0, 0)),
            pl.BlockSpec((1, _OUT), lambda b, i: (0, 0)),
        ],
        out_specs=pl.BlockSpec((1, _BI, _OUT), lambda b, i: (b, i, 0)),
        out_shape=jax.ShapeDtypeStruct((_B, _N, _OUT), jnp.float32),
        compiler_params=pltpu.CompilerParams(
            dimension_semantics=("parallel", "arbitrary")),
    )(features_batch, features_batch, adj_mats_batch, W, a_src_mat,
      a_dst_mat, bias2d)
